# Initial kernel scaffold; baseline (speedup 1.0000x reference)
import jax
import jax.numpy as jnp
from jax.experimental import pallas as pl


def kernel(predicted_class, predicted_mask, target_mask, query_idx, sample_coord, random_coord):
    raise NotImplementedError("write your pallas kernel here")


# speedup vs baseline: 29.4585x; 29.4585x over previous
import functools

import jax
import jax.numpy as jnp
from jax import lax
from jax.experimental import pallas as pl
from jax.experimental.pallas import tpu as pltpu
from jax.experimental.pallas import tpu_sc as plsc

B_, Q_, T_ = 4, 100, 10
HP_ = 112
HT_ = 256
K_ = 12544
NS_ = 37632
NIMP_ = 9408
NRAND_ = 3136
M_ = 40
PHW_ = HP_ * HP_
THW_ = HT_ * HT_
W_EOS_ = 0.1

OFF_V = 0
OFF_PRED = 37632
OFF_CB = 50176
OFF_TGT = 62720
ARENA_ = 128256

S_CHUNK = 6272
N_CHUNKS = NS_ // S_CHUNK
VPC = S_CHUNK // 16


def _bilin(arena, img_off, img_w, img_h, x, y):
    xi = (x + 1.0).astype(jnp.int32) - 1
    yi = (y + 1.0).astype(jnp.int32) - 1
    wx1 = x - xi.astype(jnp.float32)
    wy1 = y - yi.astype(jnp.float32)
    wx0 = 1.0 - wx1
    wy0 = 1.0 - wy1
    img = arena.at[pl.ds(img_off, img_w * img_h)]
    acc = jnp.zeros((16,), jnp.float32)
    for dy, wy in ((0, wy0), (1, wy1)):
        for dx, wx in ((0, wx0), (1, wx1)):
            cx = xi + dx
            cy = yi + dy
            valid = (cx >= 0) & (cx < img_w) & (cy >= 0) & (cy < img_h)
            lin = (jnp.clip(cy, 0, img_h - 1) * img_w
                   + jnp.clip(cx, 0, img_w - 1))
            val = plsc.load_gather(img, [lin])
            acc = acc + jnp.where(valid, (wy * wx) * val, 0.0)
    return acc


def _coords(arena, lane, j):
    ix = (j * 16 + lane) * 2
    cb = arena.at[pl.ds(OFF_CB, 2 * S_CHUNK)]
    sx = plsc.load_gather(cb, [ix])
    sy = plsc.load_gather(cb, [ix + 1])
    return sx, sy


def _stage_a(arena, lane, m, sc_hbm):
    def chunk(c, _):
        pltpu.sync_copy(
            sc_hbm.at[pl.ds(m * 2 * NS_ + c * 2 * S_CHUNK, 2 * S_CHUNK)],
            arena.at[pl.ds(OFF_CB, 2 * S_CHUNK)])

        def body(j, _):
            sx, sy = _coords(arena, lane, j)
            v = _bilin(arena, OFF_PRED, HP_, HP_,
                       sx * float(HP_) - 0.5, sy * float(HP_) - 0.5)
            arena[pl.ds(OFF_V + c * S_CHUNK + j * 16, 16)] = v
            return 0

        lax.fori_loop(0, VPC, body, 0)
        return 0

    lax.fori_loop(0, N_CHUNKS, chunk, 0)


def _hist_pass(arena, lane, shift, nbits, prefix, prefix_shift, k_rem):
    def clr(i, _):
        arena[pl.ds(OFF_TGT + i * 16, 16)] = jnp.zeros((16,), jnp.float32)
        return 0

    lax.fori_loop(0, 256, clr, 0)
    nb_mask = (1 << nbits) - 1
    hist = arena.at[pl.ds(OFF_TGT, 4096)]

    def body(j, _):
        v = arena[pl.ds(OFF_V + j * 16, 16)]
        u = plsc.bitcast(jnp.abs(v), jnp.int32)
        ok = lax.shift_right_logical(u, prefix_shift) == prefix
        bin_ = jnp.bitwise_and(lax.shift_right_logical(u, shift), nb_mask)
        plsc.addupdate_scatter(hist, [bin_ * 16 + lane],
                               jnp.ones((16,), jnp.float32), mask=ok)
        return 0

    lax.fori_loop(0, NS_ // 16, body, 0)

    def scan(b, carry):
        cum, bsel, kin = carry
        cnt = jnp.sum(arena[pl.ds(OFF_TGT + b * 16, 16)]).astype(jnp.int32)
        hit = (cum < k_rem) & (cum + cnt >= k_rem)
        bsel = jnp.where(hit, b, bsel)
        kin = jnp.where(hit, k_rem - cum, kin)
        return (cum + cnt, bsel, kin)

    _, bsel, kin = lax.fori_loop(0, 1 << nbits, scan,
                                 (jnp.int32(0), jnp.int32(0), jnp.int32(0)))
    return bsel, kin


def _select_threshold(arena, lane):
    k_rem = jnp.int32(NIMP_)
    prefix = jnp.int32(0)
    pshift = 31
    for shift, nbits in ((23, 8), (15, 8), (7, 8), (0, 7)):
        bsel, k_rem = _hist_pass(arena, lane, shift, nbits, prefix, pshift,
                                 k_rem)
        prefix = prefix * (1 << nbits) + bsel
        pshift = shift
    return prefix, k_rem


def _stage_d2(arena, lane, m, rc_hbm, pp_hbm, tp_hbm):
    pltpu.sync_copy(rc_hbm.at[pl.ds(m * 2 * NRAND_, 2 * NRAND_)],
                    arena.at[pl.ds(OFF_CB, 2 * NRAND_)])
    off_pr = OFF_CB + 2 * NRAND_
    off_tr = off_pr + NRAND_

    def body(j, _):
        ix = (j * 16 + lane) * 2
        cb = arena.at[pl.ds(OFF_CB, 2 * NRAND_)]
        rx = plsc.load_gather(cb, [ix])
        ry = plsc.load_gather(cb, [ix + 1])
        vp = _bilin(arena, OFF_PRED, HP_, HP_,
                    rx * float(HP_) - 0.5, ry * float(HP_) - 0.5)
        vt = _bilin(arena, OFF_TGT, HT_, HT_,
                    rx * float(HT_) - 0.5, ry * float(HT_) - 0.5)
        arena[pl.ds(off_pr + j * 16, 16)] = vp
        arena[pl.ds(off_tr + j * 16, 16)] = vt
        return 0

    lax.fori_loop(0, NRAND_ // 16, body, 0)
    pltpu.sync_copy(arena.at[pl.ds(off_pr, NRAND_)],
                    pp_hbm.at[pl.ds(m * K_ + NIMP_, NRAND_)])
    pltpu.sync_copy(arena.at[pl.ds(off_tr, NRAND_)],
                    tp_hbm.at[pl.ds(m * K_ + NIMP_, NRAND_)])


def _stage_d1(arena, lane, m, sc_hbm, pp_hbm, tp_hbm, thr, rbudget):
    def chunk(c, carry):
        pltpu.sync_copy(
            sc_hbm.at[pl.ds(m * 2 * NS_ + c * 2 * S_CHUNK, 2 * S_CHUNK)],
            arena.at[pl.ds(OFF_CB, 2 * S_CHUNK)])

        def body(j, carry2):
            off, ties = carry2
            sidx = c * S_CHUNK + j * 16
            v = arena[pl.ds(OFF_V + sidx, 16)]
            u = plsc.bitcast(jnp.abs(v), jnp.int32)
            lt = u < thr
            eq = u == thr
            eqi = eq.astype(jnp.int32)
            exc = plsc.cumsum(eqi) - eqi
            keep = lt | (eq & ((exc + ties) < rbudget))
            sx, sy = _coords(arena, lane, j)
            vt = _bilin(arena, OFF_TGT, HT_, HT_,
                        sx * float(HT_) - 0.5, sy * float(HT_) - 0.5)
            plsc.store_compressed(arena.at[pl.ds(OFF_V + off, 16)], v,
                                  mask=keep)
            plsc.store_compressed(arena.at[pl.ds(OFF_PRED + off, 16)], vt,
                                  mask=keep)
            return (off + jnp.sum(keep.astype(jnp.int32)),
                    ties + jnp.sum(eqi))

        return lax.fori_loop(0, VPC, body, carry)

    lax.fori_loop(0, N_CHUNKS, chunk, (jnp.int32(0), jnp.int32(0)))
    pltpu.sync_copy(arena.at[pl.ds(OFF_V, NIMP_)],
                    pp_hbm.at[pl.ds(m * K_, NIMP_)])
    pltpu.sync_copy(arena.at[pl.ds(OFF_PRED, NIMP_)],
                    tp_hbm.at[pl.ds(m * K_, NIMP_)])


def _sc_body(pred_hbm, tgt_hbm, ids_hbm, sc_hbm, rc_hbm, pp_hbm, tp_hbm,
             arena, ids_v):
    wid = lax.axis_index("s") * 2 + lax.axis_index("c")
    pltpu.sync_copy(ids_hbm, ids_v)
    lane = lax.iota(jnp.int32, 16)

    def row(t, _):
        m = wid + 32 * t

        @pl.when(m < M_)
        def _():
            rid = jnp.max(plsc.load_gather(ids_v, [jnp.full((16,), m, jnp.int32)]))
            pltpu.sync_copy(pred_hbm.at[pl.ds(rid * PHW_, PHW_)],
                            arena.at[pl.ds(OFF_PRED, PHW_)])
            _stage_a(arena, lane, m, sc_hbm)
            thr, rbudget = _select_threshold(arena, lane)
            pltpu.sync_copy(tgt_hbm.at[pl.ds(m * THW_, THW_)],
                            arena.at[pl.ds(OFF_TGT, THW_)])
            _stage_d2(arena, lane, m, rc_hbm, pp_hbm, tp_hbm)
            _stage_d1(arena, lane, m, sc_hbm, pp_hbm, tp_hbm, thr, rbudget)

        return 0

    lax.fori_loop(0, 2, row, 0)


_sc_kernel = functools.partial(
    pl.kernel,
    out_type=(jax.ShapeDtypeStruct((M_ * K_,), jnp.float32),
              jax.ShapeDtypeStruct((M_ * K_,), jnp.float32)),
    mesh=plsc.VectorSubcoreMesh(core_axis_name="c", subcore_axis_name="s", num_cores=2, num_subcores=16),
    scratch_types=[pltpu.VMEM((ARENA_,), jnp.float32),
                   pltpu.VMEM((48,), jnp.int32)],
    compiler_params=pltpu.CompilerParams(needs_layout_passes=False),
)(_sc_body)


def _tc_body(pp_ref, tp_ref, cls_ref, fidx_ref, out_ref):
    pp = pp_ref[...]
    tp = tp_ref[...]
    bce = (jnp.maximum(pp, 0.0) - pp * tp
           + jnp.log1p(jnp.exp(-jnp.abs(pp))))
    mask_loss = jnp.sum(bce) / (K_ * M_)
    prob = jax.nn.sigmoid(pp)
    num = 2.0 * jnp.sum(prob * tp, axis=-1, keepdims=True)
    den = (jnp.sum(prob, axis=-1, keepdims=True)
           + jnp.sum(tp, axis=-1, keepdims=True))
    dice_loss = jnp.sum(1.0 - (num + 1.0) / (den + 1.0)) / M_

    logits = cls_ref[...]
    mx = jnp.max(logits, axis=-1, keepdims=True)
    lse = mx + jnp.log(jnp.sum(jnp.exp(logits - mx), axis=-1, keepdims=True))
    logp = logits - lse
    fidx = fidx_ref[...]
    jj = lax.broadcasted_iota(jnp.int32, (B_ * Q_, 1), 0)
    is0 = jnp.any(jj == fidx, axis=1, keepdims=True)
    picked = jnp.where(is0, logp[:, 0:1], logp[:, 1:2])
    wts = jnp.where(is0, 1.0, W_EOS_)
    class_loss = -jnp.sum(wts * picked) / jnp.sum(wts)

    li = lax.broadcasted_iota(jnp.int32, (1, 128), 1)
    out_ref[...] = jnp.where(
        li == 0, class_loss,
        jnp.where(li == 1, mask_loss,
                  jnp.where(li == 2, dice_loss, 0.0)))


def kernel(predicted_class, predicted_mask, target_mask, query_idx,
           sample_coord, random_coord):
    pred_flat = predicted_mask.reshape(B_ * Q_ * PHW_)
    tgt_flat = target_mask.reshape(M_ * THW_)
    qi = query_idx.astype(jnp.int32)
    bidx = jnp.repeat(jnp.arange(B_, dtype=jnp.int32), T_)
    row_ids = bidx * Q_ + qi
    ids_pad = jnp.concatenate([row_ids, jnp.zeros((8,), jnp.int32)])
    sc_flat = sample_coord.reshape(M_ * 2 * NS_)
    rc_flat = random_coord.reshape(M_ * 2 * NRAND_)

    pp, tp = _sc_kernel(pred_flat, tgt_flat, ids_pad, sc_flat, rc_flat)
    pp = pp.reshape(M_, K_)
    tp = tp.reshape(M_, K_)

    out = pl.pallas_call(
        _tc_body,
        out_shape=jax.ShapeDtypeStruct((1, 128), jnp.float32),
    )(pp, tp, predicted_class.reshape(B_ * Q_, 2), row_ids.reshape(1, M_))
    return out[0, :3]
